# baseline (speedup 1.0000x reference)
import jax
import jax.numpy as jnp
from jax.experimental import pallas as pl

VOCAB = 1000
SEGS = 16
WIDE = SEGS * VOCAB
ROWS_PER_BLOCK = 128


def _zero_block(ids_ref, out_ref):
    out_ref[...] = jnp.zeros((ROWS_PER_BLOCK, WIDE), jnp.float32)


def kernel(input_ids) -> jnp.ndarray:
    B, L = input_ids.shape
    n = B * L
    rows = n // SEGS
    nb = rows // ROWS_PER_BLOCK
    ids = input_ids.reshape(rows, SEGS).astype(jnp.int32)
    out = pl.pallas_call(
        _zero_block,
        grid=(nb,),
        in_specs=[pl.BlockSpec((ROWS_PER_BLOCK, SEGS), lambda i: (i, 0))],
        out_specs=pl.BlockSpec((ROWS_PER_BLOCK, WIDE), lambda i: (i, 0)),
        out_shape=jax.ShapeDtypeStruct((rows, WIDE), jnp.float32),
    )(ids)
    return out.reshape(B, L, VOCAB)


# speedup vs baseline: 1.4996x; 1.4996x over previous
import jax
import jax.numpy as jnp
from jax.experimental import pallas as pl

VOCAB = 1000
ROWS_PER_BLOCK = 1024


def _onehot_block(ids_ref, out_ref):
    ids = ids_ref[0, 0, :]
    iota = jax.lax.broadcasted_iota(jnp.int32, (ROWS_PER_BLOCK, VOCAB), 1)
    out_ref[...] = (iota == ids[:, None]).astype(jnp.float32)


def kernel(input_ids) -> jnp.ndarray:
    B, L = input_ids.shape
    n = B * L
    nb = n // ROWS_PER_BLOCK
    ids_flat = input_ids.reshape(nb, 1, ROWS_PER_BLOCK).astype(jnp.int32)
    out = pl.pallas_call(
        _onehot_block,
        grid=(nb,),
        in_specs=[pl.BlockSpec((1, 1, ROWS_PER_BLOCK), lambda i: (i, 0, 0))],
        out_specs=pl.BlockSpec((ROWS_PER_BLOCK, VOCAB), lambda i: (i, 0)),
        out_shape=jax.ShapeDtypeStruct((n, VOCAB), jnp.float32),
    )(ids_flat)
    return out.reshape(B, L, VOCAB)
